# baseline (speedup 1.0000x reference)
import functools

import numpy as np
import jax
import jax.numpy as jnp
from jax import lax
from jax.experimental import pallas as pl
from jax.experimental.pallas import tpu as pltpu
from jax.experimental.pallas import tpu_sc as plsc

SWAPS = 3
TOTAL = 32768
D = 256

_R1 = np.array([
    [1488030591, 1439099953, 609311445, 260233583, 2118697808, 1156803210,
     1035656343, 1252340714, 2040732033, 1654184288, 625733951, 2086750115,
     1874956968, 2107435338, 909013543, 1372756728],
    [814496280, 34270915, 956997115, 1298601280, 1768113150, 362021218,
     1361115147, 1056098339, 573036096, 962978325, 809066367, 1194074332,
     995758540, 606323265, 1851992991, 1661132541],
    [598165367, 1415523960, 1457916550, 1099422680, 1929759519, 1650016823,
     572115305, 331872980, 355992025, 1585257322, 2054227298, 1414753250,
     442513397, 1800052159, 1325430924, 32135240],
], dtype=np.int32)
_R2 = np.array([
    [1715617077, 264418369, 1417469686, 1457313676, 1352360519, 704757104,
     204966081, 2131313276, 1215959837, 1341945816, 1932178866, 1997354769,
     745677025, 1982421356, 1148378356, 501647516],
    [2011647921, 1141977827, 233273015, 1815371096, 1213686418, 1851131719,
     1053696218, 1906738905, 1205344136, 1973623633, 1332682781, 498722935,
     1227700694, 1792697582, 654972072, 902973260],
    [3148295, 574972484, 1194890849, 831668196, 1051806027, 2105552124,
     619480870, 1217665471, 1968368069, 2036945824, 1286465655, 1900108255,
     1027825450, 1450122370, 1147306558, 449884186],
], dtype=np.int32)

_NC = 2
_NS = 16
_NW = _NC * _NS
_RPW = TOTAL // _NW
_LANES = 16
_NFIX = 2 * SWAPS * _LANES
_CHUNK = 128
_NCHUNK = _RPW // _CHUNK
_NBUF = 3

_GATHER_DNUMS = lax.GatherDimensionNumbers(
    offset_dims=(), collapsed_slice_dims=(0,), start_index_map=(0,))


def _bcast_lane(vec, j):
    idx = jnp.full((_LANES, 1), j, dtype=jnp.int32)
    return lax.gather(vec, idx, _GATHER_DNUMS, (1,),
                      mode=lax.GatherScatterMode.PROMISE_IN_BOUNDS)


def _swap_tables(r1, r2, starts, lens):
    safe = jnp.maximum(lens, 1)
    F = []
    for s in range(SWAPS):
        F.append(starts + r1[s] % safe)
        F.append(starts + r2[s] % safe)
    V = list(F)
    for s in range(SWAPS):
        v1 = V[2 * s]
        v2 = V[2 * s + 1]
        for (g, w) in ((F[2 * s], v2), (F[2 * s + 1], v1)):
            for j in range(_LANES):
                gj = _bcast_lane(g, j)
                wj = _bcast_lane(w, j)
                for t in range(2 * SWAPS):
                    V[t] = jnp.where(F[t] == gj, wj, V[t])
    return F, V


def _sc_body(tbl_hbm, flat_hbm, out_hbm,
             tbl_v, src_v, dst_v, fixrows_v, rows_v, gsem, wsem, fsem, ssem):
    wid = lax.axis_index("s") * _NC + lax.axis_index("c")
    base = wid * _RPW

    gd = [None] * _NCHUNK
    wd = [None] * _NCHUNK

    def _rd(c, b):
        return pltpu.async_copy(
            flat_hbm.at[pl.ds(base + c * _CHUNK, _CHUNK)],
            rows_v.at[b], gsem.at[b])

    for c in range(_NBUF):
        gd[c] = _rd(c, c % _NBUF)

    pltpu.sync_copy(tbl_hbm, tbl_v)
    r1 = [tbl_v[s, :] for s in range(SWAPS)]
    r2 = [tbl_v[SWAPS + s, :] for s in range(SWAPS)]
    starts = tbl_v[2 * SWAPS, :]
    lens = tbl_v[2 * SWAPS + 1, :]

    F, V = _swap_tables(r1, r2, starts, lens)

    bvec = jnp.full((_LANES,), base, dtype=jnp.int32)
    m1 = jnp.full((_LANES,), -1, dtype=jnp.int32)
    for t in range(2 * SWAPS):
        m1 = jnp.where(F[t] == bvec, V[t], m1)
    fillvec = bvec
    for j in range(_LANES):
        cj = _bcast_lane(m1, j)
        fillvec = jnp.where(cj >= 0, cj, fillvec)

    for t in range(2 * SWAPS):
        owned = (F[t] >= base) & (F[t] < base + _RPW)
        src_v[pl.ds(t * _LANES, _LANES)] = jnp.where(owned, V[t], fillvec)
        dst_v[pl.ds(t * _LANES, _LANES)] = jnp.where(owned, F[t], bvec)

    fix = pltpu.async_copy(flat_hbm.at[src_v], fixrows_v, fsem)

    for c in range(_NCHUNK):
        b = c % _NBUF
        gd[c].wait()
        wd[c] = pltpu.async_copy(rows_v.at[b],
                                 out_hbm.at[pl.ds(base + c * _CHUNK, _CHUNK)],
                                 wsem.at[b])
        if c + _NBUF < _NCHUNK:
            wd[c].wait()
            gd[c + _NBUF] = _rd(c + _NBUF, b)
    for c in range(_NCHUNK - _NBUF, _NCHUNK):
        wd[c].wait()

    fix.wait()
    pltpu.async_copy(fixrows_v, out_hbm.at[dst_v], ssem).wait()


_RTBL = np.concatenate([_R1, _R2], axis=0)


@jax.jit
def kernel(flat, cu_seqlens):
    starts = cu_seqlens[:-1]
    lens = cu_seqlens[1:] - starts
    tbl = jnp.concatenate(
        [jnp.asarray(_RTBL), starts[None, :], lens[None, :]], axis=0)
    mesh = plsc.VectorSubcoreMesh(core_axis_name="c", subcore_axis_name="s")
    run = functools.partial(
        pl.kernel,
        mesh=mesh,
        out_type=jax.ShapeDtypeStruct((TOTAL, D), jnp.float32),
        scratch_types=[
            pltpu.VMEM((2 * SWAPS + 2, _LANES), jnp.int32),
            pltpu.VMEM((_NFIX,), jnp.int32),
            pltpu.VMEM((_NFIX,), jnp.int32),
            pltpu.VMEM((_NFIX, D), jnp.float32),
            pltpu.VMEM((_NBUF, _CHUNK, D), jnp.float32),
            pltpu.SemaphoreType.DMA((_NBUF,)),
            pltpu.SemaphoreType.DMA((_NBUF,)),
            pltpu.SemaphoreType.DMA,
            pltpu.SemaphoreType.DMA,
        ],
    )(_sc_body)
    return run(tbl, flat)


# speedup vs baseline: 1.3443x; 1.3443x over previous
import functools

import numpy as np
import jax
import jax.numpy as jnp
from jax import lax
from jax.experimental import pallas as pl
from jax.experimental.pallas import tpu as pltpu
from jax.experimental.pallas import tpu_sc as plsc

SWAPS = 3
TOTAL = 32768
D = 256

_R1 = np.array([
    [1488030591, 1439099953, 609311445, 260233583, 2118697808, 1156803210,
     1035656343, 1252340714, 2040732033, 1654184288, 625733951, 2086750115,
     1874956968, 2107435338, 909013543, 1372756728],
    [814496280, 34270915, 956997115, 1298601280, 1768113150, 362021218,
     1361115147, 1056098339, 573036096, 962978325, 809066367, 1194074332,
     995758540, 606323265, 1851992991, 1661132541],
    [598165367, 1415523960, 1457916550, 1099422680, 1929759519, 1650016823,
     572115305, 331872980, 355992025, 1585257322, 2054227298, 1414753250,
     442513397, 1800052159, 1325430924, 32135240],
], dtype=np.int32)
_R2 = np.array([
    [1715617077, 264418369, 1417469686, 1457313676, 1352360519, 704757104,
     204966081, 2131313276, 1215959837, 1341945816, 1932178866, 1997354769,
     745677025, 1982421356, 1148378356, 501647516],
    [2011647921, 1141977827, 233273015, 1815371096, 1213686418, 1851131719,
     1053696218, 1906738905, 1205344136, 1973623633, 1332682781, 498722935,
     1227700694, 1792697582, 654972072, 902973260],
    [3148295, 574972484, 1194890849, 831668196, 1051806027, 2105552124,
     619480870, 1217665471, 1968368069, 2036945824, 1286465655, 1900108255,
     1027825450, 1450122370, 1147306558, 449884186],
], dtype=np.int32)

_NC = 2
_LANES = 16
_NFIX = 2 * SWAPS * _LANES
_BS = 8192
_NBLK = TOTAL // _BS

_GATHER_DNUMS = lax.GatherDimensionNumbers(
    offset_dims=(), collapsed_slice_dims=(0,), start_index_map=(0,))


def _bcast_lane(vec, j):
    idx = jnp.full((_LANES, 1), j, dtype=jnp.int32)
    return lax.gather(vec, idx, _GATHER_DNUMS, (1,),
                      mode=lax.GatherScatterMode.PROMISE_IN_BOUNDS)


def _swap_tables(r1, r2, starts, lens):
    safe = jnp.maximum(lens, 1)
    F = []
    for s in range(SWAPS):
        F.append(starts + r1[s] % safe)
        F.append(starts + r2[s] % safe)
    V = list(F)
    for s in range(SWAPS):
        v1 = V[2 * s]
        v2 = V[2 * s + 1]
        for (g, w) in ((F[2 * s], v2), (F[2 * s + 1], v1)):
            for j in range(_LANES):
                gj = _bcast_lane(g, j)
                wj = _bcast_lane(w, j)
                for t in range(2 * SWAPS):
                    V[t] = jnp.where(F[t] == gj, wj, V[t])
    return F, V


def _sc_fix_body(tbl_hbm, flat_hbm, fixdata_hbm, fixdst_hbm,
                 tbl_v, src_v, dst_v, fixrows_v, fsem):
    wid = lax.axis_index("s") * _NC + lax.axis_index("c")

    pltpu.sync_copy(tbl_hbm, tbl_v)
    r1 = [tbl_v[s, :] for s in range(SWAPS)]
    r2 = [tbl_v[SWAPS + s, :] for s in range(SWAPS)]
    starts = tbl_v[2 * SWAPS, :]
    lens = tbl_v[2 * SWAPS + 1, :]

    F, V = _swap_tables(r1, r2, starts, lens)

    for t in range(2 * SWAPS):
        src_v[pl.ds(t * _LANES, _LANES)] = V[t]
        dst_v[pl.ds(t * _LANES, _LANES)] = F[t]

    @pl.when(wid == 0)
    def _():
        pltpu.async_copy(flat_hbm.at[src_v], fixrows_v, fsem).wait()
        pltpu.sync_copy(fixrows_v, fixdata_hbm)
        pltpu.sync_copy(dst_v, fixdst_hbm)


def _tc_copy_fix_body(fixdst_ref, flat_ref, fixdata_ref, out_ref):
    i = pl.program_id(0)
    out_ref[...] = flat_ref[...]
    base = i * _BS
    for f in range(_NFIX):
        rel = fixdst_ref[f] - base

        @pl.when((rel >= 0) & (rel < _BS))
        def _():
            out_ref[pl.ds(rel, 1), :] = fixdata_ref[pl.ds(f, 1), :]


_RTBL = np.concatenate([_R1, _R2], axis=0)


@jax.jit
def kernel(flat, cu_seqlens):
    starts = cu_seqlens[:-1]
    lens = cu_seqlens[1:] - starts
    tbl = jnp.concatenate(
        [jnp.asarray(_RTBL), starts[None, :], lens[None, :]], axis=0)

    mesh = plsc.VectorSubcoreMesh(core_axis_name="c", subcore_axis_name="s")
    sc_fix = functools.partial(
        pl.kernel,
        mesh=mesh,
        out_type=(
            jax.ShapeDtypeStruct((_NFIX, D), jnp.float32),
            jax.ShapeDtypeStruct((_NFIX,), jnp.int32),
        ),
        scratch_types=[
            pltpu.VMEM((2 * SWAPS + 2, _LANES), jnp.int32),
            pltpu.VMEM((_NFIX,), jnp.int32),
            pltpu.VMEM((_NFIX,), jnp.int32),
            pltpu.VMEM((_NFIX, D), jnp.float32),
            pltpu.SemaphoreType.DMA,
        ],
    )(_sc_fix_body)
    fixdata, fixdst = sc_fix(tbl, flat)

    return pl.pallas_call(
        _tc_copy_fix_body,
        grid=(_NBLK,),
        in_specs=[
            pl.BlockSpec(memory_space=pltpu.SMEM),
            pl.BlockSpec((_BS, D), lambda i: (i, 0)),
            pl.BlockSpec((_NFIX, D), lambda i: (0, 0)),
        ],
        out_specs=pl.BlockSpec((_BS, D), lambda i: (i, 0)),
        out_shape=jax.ShapeDtypeStruct((TOTAL, D), jnp.float32),
    )(fixdst, flat, fixdata)


# speedup vs baseline: 1.5593x; 1.1600x over previous
import functools

import numpy as np
import jax
import jax.numpy as jnp
from jax import lax
from jax.experimental import pallas as pl
from jax.experimental.pallas import tpu as pltpu
from jax.experimental.pallas import tpu_sc as plsc

SWAPS = 3
TOTAL = 32768
D = 256

_R1 = np.array([
    [1488030591, 1439099953, 609311445, 260233583, 2118697808, 1156803210,
     1035656343, 1252340714, 2040732033, 1654184288, 625733951, 2086750115,
     1874956968, 2107435338, 909013543, 1372756728],
    [814496280, 34270915, 956997115, 1298601280, 1768113150, 362021218,
     1361115147, 1056098339, 573036096, 962978325, 809066367, 1194074332,
     995758540, 606323265, 1851992991, 1661132541],
    [598165367, 1415523960, 1457916550, 1099422680, 1929759519, 1650016823,
     572115305, 331872980, 355992025, 1585257322, 2054227298, 1414753250,
     442513397, 1800052159, 1325430924, 32135240],
], dtype=np.int32)
_R2 = np.array([
    [1715617077, 264418369, 1417469686, 1457313676, 1352360519, 704757104,
     204966081, 2131313276, 1215959837, 1341945816, 1932178866, 1997354769,
     745677025, 1982421356, 1148378356, 501647516],
    [2011647921, 1141977827, 233273015, 1815371096, 1213686418, 1851131719,
     1053696218, 1906738905, 1205344136, 1973623633, 1332682781, 498722935,
     1227700694, 1792697582, 654972072, 902973260],
    [3148295, 574972484, 1194890849, 831668196, 1051806027, 2105552124,
     619480870, 1217665471, 1968368069, 2036945824, 1286465655, 1900108255,
     1027825450, 1450122370, 1147306558, 449884186],
], dtype=np.int32)

_NC = 2
_LANES = 16
_NFIX = 2 * SWAPS * _LANES
_BS = 8192
_NBLK = TOTAL // _BS

_GATHER_DNUMS = lax.GatherDimensionNumbers(
    offset_dims=(), collapsed_slice_dims=(0,), start_index_map=(0,))


def _bcast_lane(vec, j):
    idx = jnp.full((_LANES, 1), j, dtype=jnp.int32)
    return lax.gather(vec, idx, _GATHER_DNUMS, (1,),
                      mode=lax.GatherScatterMode.PROMISE_IN_BOUNDS)


def _swap_tables(r1, r2, starts, lens):
    safe = jnp.maximum(lens, 1)
    F = []
    for s in range(SWAPS):
        F.append(starts + r1[s] % safe)
        F.append(starts + r2[s] % safe)
    V = list(F)
    for s in range(SWAPS):
        v1 = V[2 * s]
        v2 = V[2 * s + 1]
        for (g, w) in ((F[2 * s], v2), (F[2 * s + 1], v1)):
            for j in range(_LANES):
                gj = _bcast_lane(g, j)
                wj = _bcast_lane(w, j)
                for t in range(2 * SWAPS):
                    V[t] = jnp.where(F[t] == gj, wj, V[t])
    return F, V


def _sc_fix_body(tbl_hbm, flat_hbm, fixdata_hbm, fixdst_hbm,
                 tbl_v, src_v, dst_v, fixrows_v, fsem):
    wid = lax.axis_index("s") * _NC + lax.axis_index("c")

    pltpu.sync_copy(tbl_hbm, tbl_v)
    r1 = [tbl_v[s, :] for s in range(SWAPS)]
    r2 = [tbl_v[SWAPS + s, :] for s in range(SWAPS)]
    starts = tbl_v[2 * SWAPS, :]
    lens = tbl_v[2 * SWAPS + 1, :]

    F, V = _swap_tables(r1, r2, starts, lens)

    for t in range(2 * SWAPS):
        src_v[pl.ds(t * _LANES, _LANES)] = V[t]
        dst_v[pl.ds(t * _LANES, _LANES)] = F[t]

    @pl.when(wid == 0)
    def _():
        pltpu.async_copy(flat_hbm.at[src_v], fixrows_v, fsem).wait()
        pltpu.sync_copy(fixrows_v, fixdata_hbm)
        pltpu.sync_copy(dst_v, fixdst_hbm)


def _tc_copy_body(flat_ref, out_ref):
    out_ref[...] = flat_ref[...]


def _tc_patch_body(fixdst_ref, fixdata_ref, out_ref, patched_ref, sem):
    del out_ref
    cps = []
    for f in range(_NFIX):
        d = fixdst_ref[f]
        cp = pltpu.make_async_copy(fixdata_ref.at[pl.ds(f, 1)],
                                   patched_ref.at[pl.ds(d, 1)], sem)
        cp.start()
        cps.append(cp)
    for cp in cps:
        cp.wait()


_RTBL = np.concatenate([_R1, _R2], axis=0)


@jax.jit
def kernel(flat, cu_seqlens):
    starts = cu_seqlens[:-1]
    lens = cu_seqlens[1:] - starts
    tbl = jnp.concatenate(
        [jnp.asarray(_RTBL), starts[None, :], lens[None, :]], axis=0)

    mesh = plsc.VectorSubcoreMesh(core_axis_name="c", subcore_axis_name="s")
    sc_fix = functools.partial(
        pl.kernel,
        mesh=mesh,
        out_type=(
            jax.ShapeDtypeStruct((_NFIX, D), jnp.float32),
            jax.ShapeDtypeStruct((_NFIX,), jnp.int32),
        ),
        scratch_types=[
            pltpu.VMEM((2 * SWAPS + 2, _LANES), jnp.int32),
            pltpu.VMEM((_NFIX,), jnp.int32),
            pltpu.VMEM((_NFIX,), jnp.int32),
            pltpu.VMEM((_NFIX, D), jnp.float32),
            pltpu.SemaphoreType.DMA,
        ],
    )(_sc_fix_body)
    fixdata, fixdst = sc_fix(tbl, flat)

    copied = pl.pallas_call(
        _tc_copy_body,
        grid=(_NBLK,),
        in_specs=[pl.BlockSpec((_BS, D), lambda i: (i, 0))],
        out_specs=pl.BlockSpec((_BS, D), lambda i: (i, 0)),
        out_shape=jax.ShapeDtypeStruct((TOTAL, D), jnp.float32),
    )(flat)

    return pl.pallas_call(
        _tc_patch_body,
        in_specs=[
            pl.BlockSpec(memory_space=pltpu.SMEM),
            pl.BlockSpec(memory_space=pltpu.VMEM),
            pl.BlockSpec(memory_space=pltpu.MemorySpace.HBM),
        ],
        out_specs=pl.BlockSpec(memory_space=pltpu.MemorySpace.HBM),
        out_shape=jax.ShapeDtypeStruct((TOTAL, D), jnp.float32),
        scratch_shapes=[pltpu.SemaphoreType.DMA],
        input_output_aliases={2: 0},
    )(fixdst, fixdata, copied)


# speedup vs baseline: 1.5763x; 1.0110x over previous
import functools

import numpy as np
import jax
import jax.numpy as jnp
from jax import lax
from jax.experimental import pallas as pl
from jax.experimental.pallas import tpu as pltpu
from jax.experimental.pallas import tpu_sc as plsc

SWAPS = 3
TOTAL = 32768
D = 256

_R1 = np.array([
    [1488030591, 1439099953, 609311445, 260233583, 2118697808, 1156803210,
     1035656343, 1252340714, 2040732033, 1654184288, 625733951, 2086750115,
     1874956968, 2107435338, 909013543, 1372756728],
    [814496280, 34270915, 956997115, 1298601280, 1768113150, 362021218,
     1361115147, 1056098339, 573036096, 962978325, 809066367, 1194074332,
     995758540, 606323265, 1851992991, 1661132541],
    [598165367, 1415523960, 1457916550, 1099422680, 1929759519, 1650016823,
     572115305, 331872980, 355992025, 1585257322, 2054227298, 1414753250,
     442513397, 1800052159, 1325430924, 32135240],
], dtype=np.int32)
_R2 = np.array([
    [1715617077, 264418369, 1417469686, 1457313676, 1352360519, 704757104,
     204966081, 2131313276, 1215959837, 1341945816, 1932178866, 1997354769,
     745677025, 1982421356, 1148378356, 501647516],
    [2011647921, 1141977827, 233273015, 1815371096, 1213686418, 1851131719,
     1053696218, 1906738905, 1205344136, 1973623633, 1332682781, 498722935,
     1227700694, 1792697582, 654972072, 902973260],
    [3148295, 574972484, 1194890849, 831668196, 1051806027, 2105552124,
     619480870, 1217665471, 1968368069, 2036945824, 1286465655, 1900108255,
     1027825450, 1450122370, 1147306558, 449884186],
], dtype=np.int32)

_NC = 2
_LANES = 16
_NFIX = 2 * SWAPS * _LANES
_BS = 8192
_NBLK = TOTAL // _BS

_GATHER_DNUMS = lax.GatherDimensionNumbers(
    offset_dims=(), collapsed_slice_dims=(0,), start_index_map=(0,))


def _bcast_lane(vec, j):
    idx = jnp.full((_LANES, 1), j, dtype=jnp.int32)
    return lax.gather(vec, idx, _GATHER_DNUMS, (1,),
                      mode=lax.GatherScatterMode.PROMISE_IN_BOUNDS)


def _swap_tables(r1, r2, starts, lens):
    safe = jnp.maximum(lens, 1)
    F = []
    for s in range(SWAPS):
        F.append(starts + r1[s] % safe)
        F.append(starts + r2[s] % safe)
    V = list(F)
    for s in range(SWAPS):
        v1 = V[2 * s]
        v2 = V[2 * s + 1]
        for (g, w) in ((F[2 * s], v2), (F[2 * s + 1], v1)):
            for j in range(_LANES):
                gj = _bcast_lane(g, j)
                wj = _bcast_lane(w, j)
                for t in range(2 * SWAPS):
                    V[t] = jnp.where(F[t] == gj, wj, V[t])
    return F, V


def _sc_fix_body(rtbl_hbm, cu_hbm, flat_hbm, fixdata_hbm, fixdst_hbm,
                 tbl_v, cu_v, src_v, dst_v, fixrows_v, fsem):
    wid = lax.axis_index("s") * _NC + lax.axis_index("c")

    pltpu.sync_copy(rtbl_hbm, tbl_v)
    pltpu.sync_copy(cu_hbm.at[pl.ds(0, _LANES)], cu_v)
    r1 = [tbl_v[s, :] for s in range(SWAPS)]
    r2 = [tbl_v[SWAPS + s, :] for s in range(SWAPS)]
    starts = cu_v[...]
    iota = lax.iota(jnp.int32, _LANES)
    shift_idx = jnp.minimum(iota + 1, _LANES - 1)[:, None]
    shifted = lax.gather(starts, shift_idx, _GATHER_DNUMS, (1,),
                         mode=lax.GatherScatterMode.PROMISE_IN_BOUNDS)
    ends = jnp.where(iota == _LANES - 1, TOTAL, shifted)
    lens = ends - starts

    F, V = _swap_tables(r1, r2, starts, lens)

    for t in range(2 * SWAPS):
        src_v[pl.ds(t * _LANES, _LANES)] = V[t]
        dst_v[pl.ds(t * _LANES, _LANES)] = F[t]

    @pl.when(wid == 0)
    def _():
        pltpu.async_copy(flat_hbm.at[src_v], fixrows_v, fsem).wait()
        pltpu.sync_copy(fixrows_v, fixdata_hbm)
        pltpu.sync_copy(dst_v, fixdst_hbm)


def _tc_copy_body(flat_ref, out_ref):
    out_ref[...] = flat_ref[...]


def _tc_patch_body(fixdst_ref, fixdata_ref, out_ref, patched_ref, sem):
    del out_ref
    cps = []
    for f in range(_NFIX):
        d = fixdst_ref[f]
        cp = pltpu.make_async_copy(fixdata_ref.at[pl.ds(f, 1)],
                                   patched_ref.at[pl.ds(d, 1)], sem)
        cp.start()
        cps.append(cp)
    for cp in cps:
        cp.wait()


_RTBL = np.concatenate([_R1, _R2], axis=0)


@jax.jit
def kernel(flat, cu_seqlens):
    mesh = plsc.VectorSubcoreMesh(core_axis_name="c", subcore_axis_name="s")
    sc_fix = functools.partial(
        pl.kernel,
        mesh=mesh,
        out_type=(
            jax.ShapeDtypeStruct((_NFIX, D), jnp.float32),
            jax.ShapeDtypeStruct((_NFIX,), jnp.int32),
        ),
        scratch_types=[
            pltpu.VMEM((2 * SWAPS, _LANES), jnp.int32),
            pltpu.VMEM((_LANES,), jnp.int32),
            pltpu.VMEM((_NFIX,), jnp.int32),
            pltpu.VMEM((_NFIX,), jnp.int32),
            pltpu.VMEM((_NFIX, D), jnp.float32),
            pltpu.SemaphoreType.DMA,
        ],
    )(_sc_fix_body)
    fixdata, fixdst = sc_fix(jnp.asarray(_RTBL), cu_seqlens, flat)

    copied = pl.pallas_call(
        _tc_copy_body,
        grid=(_NBLK,),
        in_specs=[pl.BlockSpec((_BS, D), lambda i: (i, 0))],
        out_specs=pl.BlockSpec((_BS, D), lambda i: (i, 0)),
        out_shape=jax.ShapeDtypeStruct((TOTAL, D), jnp.float32),
    )(flat)

    return pl.pallas_call(
        _tc_patch_body,
        in_specs=[
            pl.BlockSpec(memory_space=pltpu.SMEM),
            pl.BlockSpec(memory_space=pltpu.VMEM),
            pl.BlockSpec(memory_space=pltpu.MemorySpace.HBM),
        ],
        out_specs=pl.BlockSpec(memory_space=pltpu.MemorySpace.HBM),
        out_shape=jax.ShapeDtypeStruct((TOTAL, D), jnp.float32),
        scratch_shapes=[pltpu.SemaphoreType.DMA],
        input_output_aliases={2: 0},
    )(fixdst, fixdata, copied)


# speedup vs baseline: 1.6822x; 1.0672x over previous
import functools

import numpy as np
import jax
import jax.numpy as jnp
from jax import lax
from jax.experimental import pallas as pl
from jax.experimental.pallas import tpu as pltpu
from jax.experimental.pallas import tpu_sc as plsc

SWAPS = 3
TOTAL = 32768
D = 256

_R1 = np.array([
    [1488030591, 1439099953, 609311445, 260233583, 2118697808, 1156803210,
     1035656343, 1252340714, 2040732033, 1654184288, 625733951, 2086750115,
     1874956968, 2107435338, 909013543, 1372756728],
    [814496280, 34270915, 956997115, 1298601280, 1768113150, 362021218,
     1361115147, 1056098339, 573036096, 962978325, 809066367, 1194074332,
     995758540, 606323265, 1851992991, 1661132541],
    [598165367, 1415523960, 1457916550, 1099422680, 1929759519, 1650016823,
     572115305, 331872980, 355992025, 1585257322, 2054227298, 1414753250,
     442513397, 1800052159, 1325430924, 32135240],
], dtype=np.int32)
_R2 = np.array([
    [1715617077, 264418369, 1417469686, 1457313676, 1352360519, 704757104,
     204966081, 2131313276, 1215959837, 1341945816, 1932178866, 1997354769,
     745677025, 1982421356, 1148378356, 501647516],
    [2011647921, 1141977827, 233273015, 1815371096, 1213686418, 1851131719,
     1053696218, 1906738905, 1205344136, 1973623633, 1332682781, 498722935,
     1227700694, 1792697582, 654972072, 902973260],
    [3148295, 574972484, 1194890849, 831668196, 1051806027, 2105552124,
     619480870, 1217665471, 1968368069, 2036945824, 1286465655, 1900108255,
     1027825450, 1450122370, 1147306558, 449884186],
], dtype=np.int32)

_NC = 2
_LANES = 16
_NFIX = 2 * SWAPS * _LANES
_BS = 8192
_NBLK = TOTAL // _BS

_GATHER_DNUMS = lax.GatherDimensionNumbers(
    offset_dims=(), collapsed_slice_dims=(0,), start_index_map=(0,))


def _bcast_lane(vec, j):
    idx = jnp.full((_LANES, 1), j, dtype=jnp.int32)
    return lax.gather(vec, idx, _GATHER_DNUMS, (1,),
                      mode=lax.GatherScatterMode.PROMISE_IN_BOUNDS)


def _swap_tables(r1, r2, starts, lens):
    safe = jnp.maximum(lens, 1)
    F = []
    for s in range(SWAPS):
        F.append(starts + r1[s] % safe)
        F.append(starts + r2[s] % safe)
    V = list(F)
    for s in range(SWAPS):
        v1 = V[2 * s]
        v2 = V[2 * s + 1]
        for (g, w) in ((F[2 * s], v2), (F[2 * s + 1], v1)):
            for j in range(_LANES):
                gj = _bcast_lane(g, j)
                wj = _bcast_lane(w, j)
                for t in range(2 * SWAPS):
                    V[t] = jnp.where(F[t] == gj, wj, V[t])
    return F, V


def _sc_fix_body(rtbl_hbm, cu_hbm, flat_hbm, fixdata_hbm, fixdst_hbm,
                 tbl_v, cu_v, src_v, dst_v, fixrows_v, fsem):
    wid = lax.axis_index("s") * _NC + lax.axis_index("c")

    pltpu.sync_copy(rtbl_hbm, tbl_v)
    pltpu.sync_copy(cu_hbm.at[pl.ds(0, _LANES)], cu_v)
    r1 = [tbl_v[s, :] for s in range(SWAPS)]
    r2 = [tbl_v[SWAPS + s, :] for s in range(SWAPS)]
    starts = cu_v[...]
    iota = lax.iota(jnp.int32, _LANES)
    shift_idx = jnp.minimum(iota + 1, _LANES - 1)[:, None]
    shifted = lax.gather(starts, shift_idx, _GATHER_DNUMS, (1,),
                         mode=lax.GatherScatterMode.PROMISE_IN_BOUNDS)
    ends = jnp.where(iota == _LANES - 1, TOTAL, shifted)
    lens = ends - starts

    F, V = _swap_tables(r1, r2, starts, lens)

    for t in range(2 * SWAPS):
        src_v[pl.ds(t * _LANES, _LANES)] = V[t]
        dst_v[pl.ds(t * _LANES, _LANES)] = F[t]

    @pl.when(wid == 0)
    def _():
        pltpu.async_copy(flat_hbm.at[src_v], fixrows_v, fsem).wait()
        pltpu.sync_copy(fixrows_v, fixdata_hbm)
        pltpu.sync_copy(dst_v, fixdst_hbm)


def _tc_copy_body(flat_ref, out_ref):
    out_ref[...] = flat_ref[...]


def _tc_patch_body(fixdst_ref, fixdata_ref, out_ref, patched_ref, sem):
    del out_ref
    cps = []
    for f in range(_NFIX):
        d = fixdst_ref[f]
        cp = pltpu.make_async_copy(fixdata_ref.at[pl.ds(f, 1)],
                                   patched_ref.at[pl.ds(d, 1)], sem)
        cp.start()
        cps.append(cp)
    for cp in cps:
        cp.wait()


_RTBL = np.concatenate([_R1, _R2], axis=0)


@jax.jit
def kernel(flat, cu_seqlens):
    mesh = plsc.VectorSubcoreMesh(core_axis_name="c", subcore_axis_name="s", num_cores=1)
    sc_fix = functools.partial(
        pl.kernel,
        mesh=mesh,
        out_type=(
            jax.ShapeDtypeStruct((_NFIX, D), jnp.float32),
            jax.ShapeDtypeStruct((_NFIX,), jnp.int32),
        ),
        scratch_types=[
            pltpu.VMEM((2 * SWAPS, _LANES), jnp.int32),
            pltpu.VMEM((_LANES,), jnp.int32),
            pltpu.VMEM((_NFIX,), jnp.int32),
            pltpu.VMEM((_NFIX,), jnp.int32),
            pltpu.VMEM((_NFIX, D), jnp.float32),
            pltpu.SemaphoreType.DMA,
        ],
    )(_sc_fix_body)
    fixdata, fixdst = sc_fix(jnp.asarray(_RTBL), cu_seqlens, flat)

    copied = pl.pallas_call(
        _tc_copy_body,
        grid=(_NBLK,),
        in_specs=[pl.BlockSpec((_BS, D), lambda i: (i, 0))],
        out_specs=pl.BlockSpec((_BS, D), lambda i: (i, 0)),
        out_shape=jax.ShapeDtypeStruct((TOTAL, D), jnp.float32),
    )(flat)

    return pl.pallas_call(
        _tc_patch_body,
        in_specs=[
            pl.BlockSpec(memory_space=pltpu.SMEM),
            pl.BlockSpec(memory_space=pltpu.VMEM),
            pl.BlockSpec(memory_space=pltpu.MemorySpace.HBM),
        ],
        out_specs=pl.BlockSpec(memory_space=pltpu.MemorySpace.HBM),
        out_shape=jax.ShapeDtypeStruct((TOTAL, D), jnp.float32),
        scratch_shapes=[pltpu.SemaphoreType.DMA],
        input_output_aliases={2: 0},
    )(fixdst, fixdata, copied)


# speedup vs baseline: 1.7455x; 1.0376x over previous
import functools

import numpy as np
import jax
import jax.numpy as jnp
from jax import lax
from jax.experimental import pallas as pl
from jax.experimental.pallas import tpu as pltpu
from jax.experimental.pallas import tpu_sc as plsc

SWAPS = 3
TOTAL = 32768
D = 256

_R1 = np.array([
    [1488030591, 1439099953, 609311445, 260233583, 2118697808, 1156803210,
     1035656343, 1252340714, 2040732033, 1654184288, 625733951, 2086750115,
     1874956968, 2107435338, 909013543, 1372756728],
    [814496280, 34270915, 956997115, 1298601280, 1768113150, 362021218,
     1361115147, 1056098339, 573036096, 962978325, 809066367, 1194074332,
     995758540, 606323265, 1851992991, 1661132541],
    [598165367, 1415523960, 1457916550, 1099422680, 1929759519, 1650016823,
     572115305, 331872980, 355992025, 1585257322, 2054227298, 1414753250,
     442513397, 1800052159, 1325430924, 32135240],
], dtype=np.int32)
_R2 = np.array([
    [1715617077, 264418369, 1417469686, 1457313676, 1352360519, 704757104,
     204966081, 2131313276, 1215959837, 1341945816, 1932178866, 1997354769,
     745677025, 1982421356, 1148378356, 501647516],
    [2011647921, 1141977827, 233273015, 1815371096, 1213686418, 1851131719,
     1053696218, 1906738905, 1205344136, 1973623633, 1332682781, 498722935,
     1227700694, 1792697582, 654972072, 902973260],
    [3148295, 574972484, 1194890849, 831668196, 1051806027, 2105552124,
     619480870, 1217665471, 1968368069, 2036945824, 1286465655, 1900108255,
     1027825450, 1450122370, 1147306558, 449884186],
], dtype=np.int32)

_NC = 2
_LANES = 16
_NFIX = 2 * SWAPS * _LANES
_BS = 8192
_NBLK = TOTAL // _BS

_GATHER_DNUMS = lax.GatherDimensionNumbers(
    offset_dims=(), collapsed_slice_dims=(0,), start_index_map=(0,))


def _bcast_lane(vec, j):
    idx = jnp.full((_LANES, 1), j, dtype=jnp.int32)
    return lax.gather(vec, idx, _GATHER_DNUMS, (1,),
                      mode=lax.GatherScatterMode.PROMISE_IN_BOUNDS)


def _swap_tables(r1, r2, starts, lens):
    safe = jnp.maximum(lens, 1)
    F = []
    for s in range(SWAPS):
        F.append(starts + r1[s] % safe)
        F.append(starts + r2[s] % safe)
    V = list(F)
    for s in range(SWAPS):
        v1 = V[2 * s]
        v2 = V[2 * s + 1]
        for (g, w) in ((F[2 * s], v2), (F[2 * s + 1], v1)):
            for j in range(_LANES):
                gj = _bcast_lane(g, j)
                wj = _bcast_lane(w, j)
                for t in range(2 * SWAPS):
                    V[t] = jnp.where(F[t] == gj, wj, V[t])
    return F, V


def _const_vec(vals, iota):
    v = jnp.full((_LANES,), int(vals[0]), dtype=jnp.int32)
    for j in range(1, _LANES):
        v = jnp.where(iota == j, int(vals[j]), v)
    return v


def _sc_fix_body(cu_hbm, flat_hbm, fixdata_hbm, fixdst_hbm,
                 cu_v, src_v, dst_v, fixrows_v, fsem):
    wid = lax.axis_index("s") * _NC + lax.axis_index("c")

    pltpu.sync_copy(cu_hbm.at[pl.ds(0, _LANES)], cu_v)
    iota0 = lax.iota(jnp.int32, _LANES)
    r1 = [_const_vec(_R1[s], iota0) for s in range(SWAPS)]
    r2 = [_const_vec(_R2[s], iota0) for s in range(SWAPS)]
    starts = cu_v[...]
    iota = lax.iota(jnp.int32, _LANES)
    shift_idx = jnp.minimum(iota + 1, _LANES - 1)[:, None]
    shifted = lax.gather(starts, shift_idx, _GATHER_DNUMS, (1,),
                         mode=lax.GatherScatterMode.PROMISE_IN_BOUNDS)
    ends = jnp.where(iota == _LANES - 1, TOTAL, shifted)
    lens = ends - starts

    F, V = _swap_tables(r1, r2, starts, lens)

    for t in range(2 * SWAPS):
        src_v[pl.ds(t * _LANES, _LANES)] = V[t]
        dst_v[pl.ds(t * _LANES, _LANES)] = F[t]

    @pl.when(wid == 0)
    def _():
        pltpu.async_copy(flat_hbm.at[src_v], fixrows_v, fsem).wait()
        pltpu.sync_copy(fixrows_v, fixdata_hbm)
        pltpu.sync_copy(dst_v, fixdst_hbm)


def _tc_copy_body(flat_ref, out_ref):
    out_ref[...] = flat_ref[...]


def _tc_patch_body(fixdst_ref, fixdata_ref, out_ref, patched_ref, sem):
    del out_ref
    cps = []
    for f in range(_NFIX):
        d = fixdst_ref[f]
        cp = pltpu.make_async_copy(fixdata_ref.at[pl.ds(f, 1)],
                                   patched_ref.at[pl.ds(d, 1)], sem)
        cp.start()
        cps.append(cp)
    for cp in cps:
        cp.wait()


_RTBL = np.concatenate([_R1, _R2], axis=0)


@jax.jit
def kernel(flat, cu_seqlens):
    mesh = plsc.VectorSubcoreMesh(core_axis_name="c", subcore_axis_name="s", num_cores=1)
    sc_fix = functools.partial(
        pl.kernel,
        mesh=mesh,
        out_type=(
            jax.ShapeDtypeStruct((_NFIX, D), jnp.float32),
            jax.ShapeDtypeStruct((_NFIX,), jnp.int32),
        ),
        scratch_types=[
            pltpu.VMEM((_LANES,), jnp.int32),
            pltpu.VMEM((_NFIX,), jnp.int32),
            pltpu.VMEM((_NFIX,), jnp.int32),
            pltpu.VMEM((_NFIX, D), jnp.float32),
            pltpu.SemaphoreType.DMA,
        ],
    )(_sc_fix_body)
    fixdata, fixdst = sc_fix(cu_seqlens, flat)

    copied = pl.pallas_call(
        _tc_copy_body,
        grid=(_NBLK,),
        in_specs=[pl.BlockSpec((_BS, D), lambda i: (i, 0))],
        out_specs=pl.BlockSpec((_BS, D), lambda i: (i, 0)),
        out_shape=jax.ShapeDtypeStruct((TOTAL, D), jnp.float32),
    )(flat)

    return pl.pallas_call(
        _tc_patch_body,
        in_specs=[
            pl.BlockSpec(memory_space=pltpu.SMEM),
            pl.BlockSpec(memory_space=pltpu.VMEM),
            pl.BlockSpec(memory_space=pltpu.MemorySpace.HBM),
        ],
        out_specs=pl.BlockSpec(memory_space=pltpu.MemorySpace.HBM),
        out_shape=jax.ShapeDtypeStruct((TOTAL, D), jnp.float32),
        scratch_shapes=[pltpu.SemaphoreType.DMA],
        input_output_aliases={2: 0},
    )(fixdst, fixdata, copied)


# speedup vs baseline: 1.7481x; 1.0015x over previous
import functools

import numpy as np
import jax
import jax.numpy as jnp
from jax import lax
from jax.experimental import pallas as pl
from jax.experimental.pallas import tpu as pltpu
from jax.experimental.pallas import tpu_sc as plsc

SWAPS = 3
TOTAL = 32768
D = 256

_R1 = np.array([
    [1488030591, 1439099953, 609311445, 260233583, 2118697808, 1156803210,
     1035656343, 1252340714, 2040732033, 1654184288, 625733951, 2086750115,
     1874956968, 2107435338, 909013543, 1372756728],
    [814496280, 34270915, 956997115, 1298601280, 1768113150, 362021218,
     1361115147, 1056098339, 573036096, 962978325, 809066367, 1194074332,
     995758540, 606323265, 1851992991, 1661132541],
    [598165367, 1415523960, 1457916550, 1099422680, 1929759519, 1650016823,
     572115305, 331872980, 355992025, 1585257322, 2054227298, 1414753250,
     442513397, 1800052159, 1325430924, 32135240],
], dtype=np.int32)
_R2 = np.array([
    [1715617077, 264418369, 1417469686, 1457313676, 1352360519, 704757104,
     204966081, 2131313276, 1215959837, 1341945816, 1932178866, 1997354769,
     745677025, 1982421356, 1148378356, 501647516],
    [2011647921, 1141977827, 233273015, 1815371096, 1213686418, 1851131719,
     1053696218, 1906738905, 1205344136, 1973623633, 1332682781, 498722935,
     1227700694, 1792697582, 654972072, 902973260],
    [3148295, 574972484, 1194890849, 831668196, 1051806027, 2105552124,
     619480870, 1217665471, 1968368069, 2036945824, 1286465655, 1900108255,
     1027825450, 1450122370, 1147306558, 449884186],
], dtype=np.int32)

_NC = 2
_LANES = 16
_NFIX = 2 * SWAPS * _LANES
_BS = 8192
_NBLK = TOTAL // _BS

_GATHER_DNUMS = lax.GatherDimensionNumbers(
    offset_dims=(), collapsed_slice_dims=(0,), start_index_map=(0,))


def _bcast_lane(vec, j):
    idx = jnp.full((_LANES, 1), j, dtype=jnp.int32)
    return lax.gather(vec, idx, _GATHER_DNUMS, (1,),
                      mode=lax.GatherScatterMode.PROMISE_IN_BOUNDS)


def _swap_tables(r1, r2, starts, lens):
    safe = jnp.maximum(lens, 1)
    F = []
    for s in range(SWAPS):
        F.append(starts + r1[s] % safe)
        F.append(starts + r2[s] % safe)
    V = list(F)
    for s in range(SWAPS):
        v1 = V[2 * s]
        v2 = V[2 * s + 1]
        for (g, w) in ((F[2 * s], v2), (F[2 * s + 1], v1)):
            def _lane_step(j, vs, g=g, w=w):
                gj = _bcast_lane(g, j)
                wj = _bcast_lane(w, j)
                return tuple(jnp.where(F[t] == gj, wj, vs[t])
                             for t in range(2 * SWAPS))

            V = list(lax.fori_loop(0, _LANES, _lane_step, tuple(V)))
    return F, V


def _const_vec(vals, iota):
    v = jnp.full((_LANES,), int(vals[0]), dtype=jnp.int32)
    for j in range(1, _LANES):
        v = jnp.where(iota == j, int(vals[j]), v)
    return v


def _sc_fix_body(cu_hbm, flat_hbm, fixdata_hbm, fixdst_hbm,
                 cu_v, src_v, dst_v, fixrows_v, fsem):
    wid = lax.axis_index("s") * _NC + lax.axis_index("c")

    pltpu.sync_copy(cu_hbm.at[pl.ds(0, _LANES)], cu_v)
    iota0 = lax.iota(jnp.int32, _LANES)
    r1 = [_const_vec(_R1[s], iota0) for s in range(SWAPS)]
    r2 = [_const_vec(_R2[s], iota0) for s in range(SWAPS)]
    starts = cu_v[...]
    iota = lax.iota(jnp.int32, _LANES)
    shift_idx = jnp.minimum(iota + 1, _LANES - 1)[:, None]
    shifted = lax.gather(starts, shift_idx, _GATHER_DNUMS, (1,),
                         mode=lax.GatherScatterMode.PROMISE_IN_BOUNDS)
    ends = jnp.where(iota == _LANES - 1, TOTAL, shifted)
    lens = ends - starts

    F, V = _swap_tables(r1, r2, starts, lens)

    for t in range(2 * SWAPS):
        src_v[pl.ds(t * _LANES, _LANES)] = V[t]
        dst_v[pl.ds(t * _LANES, _LANES)] = F[t]

    @pl.when(wid == 0)
    def _():
        pltpu.async_copy(flat_hbm.at[src_v], fixrows_v, fsem).wait()
        pltpu.sync_copy(fixrows_v, fixdata_hbm)
        pltpu.sync_copy(dst_v, fixdst_hbm)


def _tc_copy_body(flat_ref, out_ref):
    out_ref[...] = flat_ref[...]


def _tc_patch_body(fixdst_ref, fixdata_ref, out_ref, patched_ref, sem):
    del out_ref
    cps = []
    for f in range(_NFIX):
        d = fixdst_ref[f]
        cp = pltpu.make_async_copy(fixdata_ref.at[pl.ds(f, 1)],
                                   patched_ref.at[pl.ds(d, 1)], sem)
        cp.start()
        cps.append(cp)
    for cp in cps:
        cp.wait()


_RTBL = np.concatenate([_R1, _R2], axis=0)


@jax.jit
def kernel(flat, cu_seqlens):
    mesh = plsc.VectorSubcoreMesh(core_axis_name="c", subcore_axis_name="s", num_cores=1)
    sc_fix = functools.partial(
        pl.kernel,
        mesh=mesh,
        out_type=(
            jax.ShapeDtypeStruct((_NFIX, D), jnp.float32),
            jax.ShapeDtypeStruct((_NFIX,), jnp.int32),
        ),
        scratch_types=[
            pltpu.VMEM((_LANES,), jnp.int32),
            pltpu.VMEM((_NFIX,), jnp.int32),
            pltpu.VMEM((_NFIX,), jnp.int32),
            pltpu.VMEM((_NFIX, D), jnp.float32),
            pltpu.SemaphoreType.DMA,
        ],
    )(_sc_fix_body)
    fixdata, fixdst = sc_fix(cu_seqlens, flat)

    copied = pl.pallas_call(
        _tc_copy_body,
        grid=(_NBLK,),
        in_specs=[pl.BlockSpec((_BS, D), lambda i: (i, 0))],
        out_specs=pl.BlockSpec((_BS, D), lambda i: (i, 0)),
        out_shape=jax.ShapeDtypeStruct((TOTAL, D), jnp.float32),
    )(flat)

    return pl.pallas_call(
        _tc_patch_body,
        in_specs=[
            pl.BlockSpec(memory_space=pltpu.SMEM),
            pl.BlockSpec(memory_space=pltpu.VMEM),
            pl.BlockSpec(memory_space=pltpu.MemorySpace.HBM),
        ],
        out_specs=pl.BlockSpec(memory_space=pltpu.MemorySpace.HBM),
        out_shape=jax.ShapeDtypeStruct((TOTAL, D), jnp.float32),
        scratch_shapes=[pltpu.SemaphoreType.DMA],
        input_output_aliases={2: 0},
    )(fixdst, fixdata, copied)
